# Initial kernel scaffold; baseline (speedup 1.0000x reference)
import jax
import jax.numpy as jnp
from jax.experimental import pallas as pl


def kernel(zi, zj, edge_index, W1, b1, W2, b2, W3, b3):
    raise NotImplementedError("write your pallas kernel here")


# speedup vs baseline: 3.5775x; 3.5775x over previous
import functools

import jax
import jax.numpy as jnp
from jax import lax
from jax.experimental import pallas as pl
from jax.experimental.pallas import tpu as pltpu
from jax.experimental.pallas import tpu_sc as plsc

N_NODES = 10000
N_EDGES = 320000
D_FEAT = 128
H1 = 32
H2 = 16

NUM_CORES = 2
NUM_SUBCORES = 16
NW = NUM_CORES * NUM_SUBCORES

EDGES_PAD = 327680
PER_W = EDGES_PAD // NW
CHUNK = 1024
NCHUNK = PER_W // CHUNK
IDX_ROWS = CHUNK // 128

_HI = lax.Precision.HIGHEST


def _prep_body(zi_ref, zj_ref, w1a_ref, w1b_ref, b1_ref, a_ref, b_ref):
    a_ref[...] = (
        jnp.dot(zi_ref[...], w1a_ref[...], precision=_HI) + b1_ref[...]
    )
    b_ref[...] = jnp.dot(zj_ref[...], w1b_ref[...], precision=_HI)


def _gather_body(a_hbm, b_hbm, src_hbm, dst_hbm, g1_hbm, g2_hbm,
                 src_v, dst_v, rows_a, rows_b, sem_a, sem_b):
    wid = lax.axis_index("s") * NUM_CORES + lax.axis_index("c")
    base0 = wid * PER_W

    def body(i, carry):
        base = base0 + i * CHUNK
        pltpu.sync_copy(src_hbm.at[pl.ds(base // 128, IDX_ROWS)], src_v)
        pltpu.sync_copy(dst_hbm.at[pl.ds(base // 128, IDX_ROWS)], dst_v)
        for j in range(IDX_ROWS):
            pltpu.async_copy(
                a_hbm.at[src_v.at[j]], rows_a.at[pl.ds(j * 128, 128)], sem_a)
            pltpu.async_copy(
                b_hbm.at[dst_v.at[j]], rows_b.at[pl.ds(j * 128, 128)], sem_b)
        for j in range(IDX_ROWS):
            pltpu.make_async_copy(
                a_hbm.at[src_v.at[j]], rows_a.at[pl.ds(j * 128, 128)],
                sem_a).wait()
            pltpu.make_async_copy(
                b_hbm.at[dst_v.at[j]], rows_b.at[pl.ds(j * 128, 128)],
                sem_b).wait()
        pltpu.sync_copy(rows_a, g1_hbm.at[pl.ds(base, CHUNK)])
        pltpu.sync_copy(rows_b, g2_hbm.at[pl.ds(base, CHUNK)])
        return carry

    lax.fori_loop(0, NCHUNK, body, 0)


BLK4 = 2048


def _mlp_body(g1_ref, g2_ref, w2_ref, b2_ref, w3_ref, b3_ref, o_ref):
    h1 = jnp.maximum(g1_ref[...] + g2_ref[...], 0.0)
    h2 = jnp.maximum(jnp.dot(h1, w2_ref[...], precision=_HI) + b2_ref[...], 0.0)
    z = jnp.dot(h2, w3_ref[...], precision=_HI) + b3_ref[...]
    o_ref[...] = 1.0 / (1.0 + jnp.exp(-z))


def kernel(zi, zj, edge_index, W1, b1, W2, b2, W3, b3):
    w1a = W1[:, :D_FEAT].T
    w1b = W1[:, D_FEAT:].T
    b1r = b1.reshape(1, H1)
    eye4 = jnp.eye(4, dtype=jnp.float32)
    w2rep = jnp.kron(eye4, W2.T)
    b2rep = jnp.tile(b2, 4).reshape(1, 4 * H2)
    w3rep = jnp.kron(eye4, W3.T)
    b3rep = jnp.tile(b3, 4).reshape(1, 4)
    src = jnp.pad(edge_index[0], (0, EDGES_PAD - N_EDGES))
    dst = jnp.pad(edge_index[1], (0, EDGES_PAD - N_EDGES))
    src2 = src.reshape(EDGES_PAD // 128, 128)
    dst2 = dst.reshape(EDGES_PAD // 128, 128)

    a_proj, b_proj = pl.pallas_call(
        _prep_body,
        out_shape=[
            jax.ShapeDtypeStruct((N_NODES, H1), jnp.float32),
            jax.ShapeDtypeStruct((N_NODES, H1), jnp.float32),
        ],
    )(zi, zj, w1a, w1b, b1r)

    mesh = plsc.VectorSubcoreMesh(core_axis_name="c", subcore_axis_name="s")
    gather = functools.partial(
        pl.kernel,
        mesh=mesh,
        out_type=[
            jax.ShapeDtypeStruct((EDGES_PAD, H1), jnp.float32),
            jax.ShapeDtypeStruct((EDGES_PAD, H1), jnp.float32),
        ],
        scratch_types=[
            pltpu.VMEM((IDX_ROWS, 128), jnp.int32),
            pltpu.VMEM((IDX_ROWS, 128), jnp.int32),
            pltpu.VMEM((CHUNK, H1), jnp.float32),
            pltpu.VMEM((CHUNK, H1), jnp.float32),
            pltpu.SemaphoreType.DMA,
            pltpu.SemaphoreType.DMA,
        ],
        compiler_params=pltpu.CompilerParams(use_tc_tiling_on_sc=False),
    )(_gather_body)
    g1, g2 = gather(a_proj, b_proj, src2, dst2)

    g1p = g1.reshape(EDGES_PAD // 4, 4 * H1)
    g2p = g2.reshape(EDGES_PAD // 4, 4 * H1)
    out = pl.pallas_call(
        _mlp_body,
        grid=(EDGES_PAD // 4 // BLK4,),
        in_specs=[
            pl.BlockSpec((BLK4, 4 * H1), lambda i: (i, 0)),
            pl.BlockSpec((BLK4, 4 * H1), lambda i: (i, 0)),
            pl.BlockSpec((4 * H1, 4 * H2), lambda i: (0, 0)),
            pl.BlockSpec((1, 4 * H2), lambda i: (0, 0)),
            pl.BlockSpec((4 * H2, 4), lambda i: (0, 0)),
            pl.BlockSpec((1, 4), lambda i: (0, 0)),
        ],
        out_specs=pl.BlockSpec((BLK4, 4), lambda i: (i, 0)),
        out_shape=jax.ShapeDtypeStruct((EDGES_PAD // 4, 4), jnp.float32),
    )(g1p, g2p, w2rep, b2rep, w3rep, b3rep)

    return out.reshape(EDGES_PAD, 1)[:N_EDGES]


# speedup vs baseline: 5.2969x; 1.4806x over previous
import functools

import jax
import jax.numpy as jnp
from jax import lax
from jax.experimental import pallas as pl
from jax.experimental.pallas import tpu as pltpu
from jax.experimental.pallas import tpu_sc as plsc

N_NODES = 10000
N_EDGES = 320000
D_FEAT = 128
H1 = 32
H2 = 16

NUM_CORES = 2
NUM_SUBCORES = 16
NW = NUM_CORES * NUM_SUBCORES

EDGES_PAD = 327680
PER_W = EDGES_PAD // NW
CHUNK = 1024
NCHUNK = PER_W // CHUNK
IDX_ROWS = CHUNK // 128
IDXW = PER_W // 128

_HI = lax.Precision.HIGHEST


def _prep_body(zi_ref, zj_ref, w1a_ref, w1b_ref, b1_ref, a_ref, b_ref):
    a = jnp.dot(zi_ref[...], w1a_ref[...], precision=_HI) + b1_ref[...]
    b = jnp.dot(zj_ref[...], w1b_ref[...], precision=_HI)
    a_ref[...] = a.astype(jnp.bfloat16)
    b_ref[...] = b.astype(jnp.bfloat16)


IDX_BLK_COLS = 32768


def _idxprep_body(ei_ref, src_ref, dst_ref):
    src_ref[...] = jnp.clip(ei_ref[0:1, :], 0, N_NODES - 1)
    dst_ref[...] = jnp.clip(ei_ref[1:2, :], 0, N_NODES - 1)


def _gather_body(a_hbm, b_hbm, src_hbm, dst_hbm, g1_hbm, g2_hbm,
                 idx_s, idx_d, rows_a0, rows_b0, rows_a1, rows_b1,
                 sem_a, sem_b, sem_w):
    wid = lax.axis_index("s") * NUM_CORES + lax.axis_index("c")
    base0 = wid * PER_W
    row0 = wid * IDXW

    pltpu.sync_copy(src_hbm.at[pl.ds(row0, IDXW)], idx_s)
    pltpu.sync_copy(dst_hbm.at[pl.ds(row0, IDXW)], idx_d)

    bufs = ((rows_a0, rows_b0), (rows_a1, rows_b1))
    pending_g = {}
    pending_w = {}

    def issue_gathers(c):
        ra, rb = bufs[c % 2]
        ds_ = []
        for j in range(IDX_ROWS):
            ds_.append(pltpu.async_copy(
                a_hbm.at[idx_s.at[c * IDX_ROWS + j]],
                ra.at[pl.ds(j * 128, 128)], sem_a))
            ds_.append(pltpu.async_copy(
                b_hbm.at[idx_d.at[c * IDX_ROWS + j]],
                rb.at[pl.ds(j * 128, 128)], sem_b))
        pending_g[c] = ds_

    def issue_writes(c):
        ra, rb = bufs[c % 2]
        base = base0 + c * CHUNK
        pending_w[c] = [
            pltpu.async_copy(ra, g1_hbm.at[pl.ds(base, CHUNK)], sem_w),
            pltpu.async_copy(rb, g2_hbm.at[pl.ds(base, CHUNK)], sem_w),
        ]

    for c in range(NCHUNK):
        if c - 2 in pending_w:
            for d_ in pending_w.pop(c - 2):
                d_.wait()
        issue_gathers(c)
        if c - 1 in pending_g:
            for d_ in pending_g.pop(c - 1):
                d_.wait()
            issue_writes(c - 1)
    for d_ in pending_g.pop(NCHUNK - 1):
        d_.wait()
    issue_writes(NCHUNK - 1)
    for c in sorted(pending_w):
        for d_ in pending_w[c]:
            d_.wait()


PACK = 32
BLKP = 256


def _mlp_body(g1_ref, g2_ref, w2_ref, b2_ref, w3_ref, b3_ref, o_ref):
    h1 = jnp.maximum(g1_ref[...] + g2_ref[...], jnp.bfloat16(0))
    h2 = jnp.dot(h1, w2_ref[...], preferred_element_type=jnp.float32)
    h2 = jnp.maximum(h2 + b2_ref[...], 0.0).astype(jnp.bfloat16)
    z = jnp.dot(h2, w3_ref[...], preferred_element_type=jnp.float32)
    z = z + b3_ref[...]
    o_ref[...] = 1.0 / (1.0 + jnp.exp(-z))


def kernel(zi, zj, edge_index, W1, b1, W2, b2, W3, b3):
    w1a = W1[:, :D_FEAT].T
    w1b = W1[:, D_FEAT:].T
    b1r = b1.reshape(1, H1)
    eyep = jnp.eye(32, dtype=jnp.float32)
    w2rep = jnp.kron(eyep, W2.T).astype(jnp.bfloat16)
    b2rep = jnp.tile(b2, 32).reshape(1, 32 * H2)
    w3rep = jnp.kron(eyep, W3.T).astype(jnp.bfloat16)
    b3rep = jnp.tile(b3, 32).reshape(1, 32)

    a_proj, b_proj = pl.pallas_call(
        _prep_body,
        out_shape=[
            jax.ShapeDtypeStruct((N_NODES, H1), jnp.bfloat16),
            jax.ShapeDtypeStruct((N_NODES, H1), jnp.bfloat16),
        ],
    )(zi, zj, w1a, w1b, b1r)

    srcp, dstp = pl.pallas_call(
        _idxprep_body,
        grid=(EDGES_PAD // IDX_BLK_COLS,),
        in_specs=[pl.BlockSpec((2, IDX_BLK_COLS), lambda i: (0, i))],
        out_specs=[
            pl.BlockSpec((1, IDX_BLK_COLS), lambda i: (0, i)),
            pl.BlockSpec((1, IDX_BLK_COLS), lambda i: (0, i)),
        ],
        out_shape=[
            jax.ShapeDtypeStruct((1, EDGES_PAD), jnp.int32),
            jax.ShapeDtypeStruct((1, EDGES_PAD), jnp.int32),
        ],
    )(edge_index)
    srcp = srcp.reshape(EDGES_PAD // 128, 128)
    dstp = dstp.reshape(EDGES_PAD // 128, 128)

    mesh = plsc.VectorSubcoreMesh(core_axis_name="c", subcore_axis_name="s")
    gather = functools.partial(
        pl.kernel,
        mesh=mesh,
        out_type=[
            jax.ShapeDtypeStruct((EDGES_PAD, H1), jnp.bfloat16),
            jax.ShapeDtypeStruct((EDGES_PAD, H1), jnp.bfloat16),
        ],
        scratch_types=[
            pltpu.VMEM((IDXW, 128), jnp.int32),
            pltpu.VMEM((IDXW, 128), jnp.int32),
            pltpu.VMEM((CHUNK, H1), jnp.bfloat16),
            pltpu.VMEM((CHUNK, H1), jnp.bfloat16),
            pltpu.VMEM((CHUNK, H1), jnp.bfloat16),
            pltpu.VMEM((CHUNK, H1), jnp.bfloat16),
            pltpu.SemaphoreType.DMA,
            pltpu.SemaphoreType.DMA,
            pltpu.SemaphoreType.DMA,
        ],
        compiler_params=pltpu.CompilerParams(use_tc_tiling_on_sc=False),
    )(_gather_body)
    g1, g2 = gather(a_proj, b_proj, srcp, dstp)

    g1p = g1.reshape(EDGES_PAD // PACK, PACK * H1)
    g2p = g2.reshape(EDGES_PAD // PACK, PACK * H1)
    out = pl.pallas_call(
        _mlp_body,
        grid=(EDGES_PAD // PACK // BLKP,),
        in_specs=[
            pl.BlockSpec((BLKP, PACK * H1), lambda i: (i, 0)),
            pl.BlockSpec((BLKP, PACK * H1), lambda i: (i, 0)),
            pl.BlockSpec((PACK * H1, PACK * H2), lambda i: (0, 0)),
            pl.BlockSpec((1, PACK * H2), lambda i: (0, 0)),
            pl.BlockSpec((PACK * H2, PACK), lambda i: (0, 0)),
            pl.BlockSpec((1, PACK), lambda i: (0, 0)),
        ],
        out_specs=pl.BlockSpec((BLKP, PACK), lambda i: (i, 0)),
        out_shape=jax.ShapeDtypeStruct((EDGES_PAD // PACK, PACK), jnp.float32),
    )(g1p, g2p, w2rep, b2rep, w3rep, b3rep)

    return out.reshape(EDGES_PAD)[:N_EDGES].reshape(N_EDGES, 1)
